# Initial kernel scaffold; baseline (speedup 1.0000x reference)
import jax
import jax.numpy as jnp
from jax.experimental import pallas as pl


def kernel(x_q, x_kv, Wq, Wq_dw, Wkv, Wkv_dw, Wproj, temperature, a1, a2, a3, a4):
    raise NotImplementedError("write your pallas kernel here")


# speedup vs baseline: 3.7967x; 3.7967x over previous
import functools

import jax
import jax.numpy as jnp
from jax.experimental import pallas as pl
from jax.experimental.pallas import tpu as pltpu

_HEADS = 8
_KKS = (12, 16, 18, 19)
_CHUNK = 3584


def _bf(x):
    return x.astype(jnp.bfloat16)


def _conv1x1_kernel(xq_ref, xkv_ref, wq_ref, wkv_ref, yq_ref, ykv_ref):
    yq_ref[...] = _bf(jnp.dot(_bf(wq_ref[...]), _bf(xq_ref[...]),
                              preferred_element_type=jnp.float32))
    ykv_ref[...] = _bf(jnp.dot(_bf(wkv_ref[...]), _bf(xkv_ref[...]),
                               preferred_element_type=jnp.float32))


def _col_masks(w_img, ch_size):
    col = jax.lax.broadcasted_iota(jnp.int32, (1, ch_size), 1) % w_img
    return col != 0, col != (w_img - 1)


def _dw_chunk(y_ref, taps, base, left_ok, right_ok, w_img, n, ch_size):
    acc = None
    for i in (-1, 0, 1):
        for j in (-1, 0, 1):
            t = (i + 1) * 3 + (j + 1)
            start = base + i * w_img + j
            lo = max(0, start)
            hi = min(n, start + ch_size)
            s = y_ref[:, lo:hi]
            padl = lo - start
            padr = (start + ch_size) - hi
            if padl or padr:
                s = jnp.pad(s, ((0, 0), (padl, padr)))
            s = s.astype(jnp.float32)
            if j == -1:
                s = jnp.where(left_ok, s, 0.0)
            elif j == 1:
                s = jnp.where(right_ok, s, 0.0)
            term = taps[:, t][:, None] * s
            acc = term if acc is None else acc + term
    return acc


def _dwgram_kernel(yq_ref, yk_ref, wq_dw_ref, wk_dw_ref, gss_ref,
                   *, w_img, n):
    left_ok, right_ok = _col_masks(w_img, _CHUNK)
    c = yq_ref.shape[0]
    tq = wq_dw_ref[...]
    tk = wk_dw_ref[...]
    nch = n // _CHUNK
    sq = jnp.zeros((1, c), jnp.float32)
    sk = jnp.zeros((1, c), jnp.float32)
    for ch in range(nch):
        base = ch * _CHUNK
        q_ch = _dw_chunk(yq_ref, tq, base, left_ok, right_ok, w_img, n, _CHUNK)
        k_ch = _dw_chunk(yk_ref, tk, base, left_ok, right_ok, w_img, n, _CHUNK)
        sq = sq + jnp.sum(q_ch * q_ch, axis=1)[None, :]
        sk = sk + jnp.sum(k_ch * k_ch, axis=1)[None, :]
    nq = jnp.maximum(jnp.sqrt(sq), 1e-12).reshape(c, 1)
    nk = jnp.maximum(jnp.sqrt(sk), 1e-12).reshape(c, 1)
    g = jnp.zeros((c, c), jnp.float32)
    for ch in range(nch):
        base = ch * _CHUNK
        q_ch = _dw_chunk(yq_ref, tq, base, left_ok, right_ok, w_img, n, _CHUNK)
        k_ch = _dw_chunk(yk_ref, tk, base, left_ok, right_ok, w_img, n, _CHUNK)
        g = g + jax.lax.dot_general(_bf(q_ch / nq), _bf(k_ch / nk),
                                    (((1,), (1,)), ((), ())),
                                    preferred_element_type=jnp.float32)
    pad_rows = jnp.zeros((32 - c - 2, c), dtype=jnp.float32)
    gss_ref[...] = jnp.concatenate([g, sq, sk, pad_rows], axis=0)[None]


def _dwv_kernel(yv_ref, wv_dw_ref, v_ref, *, w_img, n):
    left_ok, right_ok = _col_masks(w_img, _CHUNK)
    tv = wv_dw_ref[...]
    for ch in range(n // _CHUNK):
        base = ch * _CHUNK
        v_ref[:, base:base + _CHUNK] = _bf(_dw_chunk(
            yv_ref, tv, base, left_ok, right_ok, w_img, n, _CHUNK))


def _maskout_kernel(gss_ref, temp_ref, a_ref, wproj_ref, v_ref, out_ref,
                    weff_ref, *, heads, c):
    @pl.when(pl.program_id(0) == 0)
    def _():
        gss = gss_ref[...]
        t = temp_ref[...][:, 0][:, None, None]
        attn = gss[:, :c, :] * t
        aj = attn[:, :, :, None]
        ajp = attn[:, :, None, :]
        jidx = jax.lax.broadcasted_iota(jnp.int32, (1, 1, c, c), 2)
        jpidx = jax.lax.broadcasted_iota(jnp.int32, (1, 1, c, c), 3)
        beats = jnp.where((ajp > aj) | ((ajp == aj) & (jpidx < jidx)), 1.0, 0.0)
        rank = jnp.sum(beats, axis=-1)
        acc = jnp.zeros_like(attn)
        for m, kk in enumerate(_KKS):
            masked = jnp.where(rank < kk, attn, -1e30)
            mx = jnp.max(masked, axis=-1, keepdims=True)
            e = jnp.exp(masked - mx)
            sm = e / jnp.sum(e, axis=-1, keepdims=True)
            acc = acc + a_ref[0, m] * sm
        wproj = wproj_ref[...]
        cols = []
        for h in range(heads):
            cols.append(jnp.dot(wproj[:, h * c:(h + 1) * c], acc[h],
                                preferred_element_type=jnp.float32,
                                precision=jax.lax.Precision.HIGHEST))
        weff_ref[...] = jnp.concatenate(cols, axis=1)

    out_ref[...] = jnp.dot(_bf(weff_ref[...]), v_ref[...],
                           preferred_element_type=jnp.float32)


def kernel(x_q, x_kv, Wq, Wq_dw, Wkv, Wkv_dw, Wproj, temperature, a1, a2, a3, a4):
    b, dim, h_img, w_img = x_q.shape
    n = h_img * w_img
    heads = _HEADS
    c = dim // heads

    xq = x_q.reshape(dim, n)
    xkv = x_kv.reshape(dim, n)
    wq = Wq.reshape(dim, dim)
    wkv = Wkv.reshape(2 * dim, dim)
    wq_dw = Wq_dw.reshape(dim, 9)
    wkv_dw = Wkv_dw.reshape(2 * dim, 9)
    wproj = Wproj.reshape(dim, dim)
    temp = jnp.broadcast_to(temperature.reshape(heads, 1), (heads, c))
    a_arr = jnp.broadcast_to(
        jnp.concatenate([a1, a2, a3, a4]).reshape(1, 4), (8, 4))

    tiles = 14
    t = n // tiles

    yq, ykv = pl.pallas_call(
        _conv1x1_kernel,
        grid=(tiles,),
        in_specs=[
            pl.BlockSpec((dim, t), lambda i: (0, i)),
            pl.BlockSpec((dim, t), lambda i: (0, i)),
            pl.BlockSpec((dim, dim), lambda i: (0, 0)),
            pl.BlockSpec((2 * dim, dim), lambda i: (0, 0)),
        ],
        out_specs=[
            pl.BlockSpec((dim, t), lambda i: (0, i)),
            pl.BlockSpec((2 * dim, t), lambda i: (0, i)),
        ],
        out_shape=[
            jax.ShapeDtypeStruct((dim, n), jnp.bfloat16),
            jax.ShapeDtypeStruct((2 * dim, n), jnp.bfloat16),
        ],
    )(xq, xkv, wq, wkv)

    gss = pl.pallas_call(
        functools.partial(_dwgram_kernel, w_img=w_img, n=n),
        grid=(heads,),
        in_specs=[
            pl.BlockSpec((c, n), lambda i: (i, 0)),
            pl.BlockSpec((c, n), lambda i: (i, 0)),
            pl.BlockSpec((c, 9), lambda i: (i, 0)),
            pl.BlockSpec((c, 9), lambda i: (i, 0)),
        ],
        out_specs=pl.BlockSpec((1, 32, c), lambda i: (i, 0, 0)),
        out_shape=jax.ShapeDtypeStruct((heads, 32, c), jnp.float32),
    )(yq, ykv, wq_dw, wkv_dw)

    v_arr = pl.pallas_call(
        functools.partial(_dwv_kernel, w_img=w_img, n=n),
        grid=(heads,),
        in_specs=[
            pl.BlockSpec((c, n), lambda i: (i + heads, 0)),
            pl.BlockSpec((c, 9), lambda i: (i + heads, 0)),
        ],
        out_specs=pl.BlockSpec((c, n), lambda i: (i, 0)),
        out_shape=jax.ShapeDtypeStruct((dim, n), jnp.bfloat16),
    )(ykv, wkv_dw)

    out = pl.pallas_call(
        functools.partial(_maskout_kernel, heads=heads, c=c),
        grid=(tiles,),
        in_specs=[
            pl.BlockSpec((heads, 32, c), lambda i: (0, 0, 0)),
            pl.BlockSpec((heads, c), lambda i: (0, 0)),
            pl.BlockSpec((8, 4), lambda i: (0, 0)),
            pl.BlockSpec((dim, dim), lambda i: (0, 0)),
            pl.BlockSpec((dim, t), lambda i: (0, i)),
        ],
        out_specs=pl.BlockSpec((dim, t), lambda i: (0, i)),
        out_shape=jax.ShapeDtypeStruct((dim, n), jnp.float32),
        scratch_shapes=[pltpu.VMEM((dim, dim), jnp.float32)],
    )(gss, temp, a_arr, wproj, v_arr)

    return out.reshape(b, dim, h_img, w_img)
